# Initial kernel scaffold; baseline (speedup 1.0000x reference)
import jax
import jax.numpy as jnp
from jax.experimental import pallas as pl


def kernel(features, feature_graph_index, W_m, b_m, W_ih, W_hh, b_ih, b_hh):
    raise NotImplementedError("write your pallas kernel here")


# speedup vs baseline: 1.0076x; 1.0076x over previous
import functools
import jax
import jax.numpy as jnp
from jax import lax
from jax.experimental import pallas as pl
from jax.experimental.pallas import tpu as pltpu

_TN = 256
_GR = 1000
_LOOPS = 3
_EPSV = 1e-07
_NEG = -1e30


def _lstm_body(h_ref, c_ref, ro_ref, so_ref, wa_ref, wb_ref, bs_ref,
               hn_ref, cn_ref):
    h = h_ref[...]
    c = c_ref[...]
    s = so_ref[:, :1]
    r = ro_ref[...] / (s + _EPSV)
    gates = (jnp.dot(h, wa_ref[...], preferred_element_type=jnp.float32)
             + jnp.dot(r, wb_ref[...], preferred_element_type=jnp.float32)
             + bs_ref[...])
    nh = wa_ref.shape[1] // 4
    gi = gates[:, 0 * nh:1 * nh]
    gf = gates[:, 1 * nh:2 * nh]
    gg = gates[:, 2 * nh:3 * nh]
    go = gates[:, 3 * nh:4 * nh]
    i_g = jax.nn.sigmoid(gi)
    f_g = jax.nn.sigmoid(gf)
    g_g = jnp.tanh(gg)
    o_g = jax.nn.sigmoid(go)
    c_new = f_g * c + i_g * g_g
    hn_ref[...] = o_g * jnp.tanh(c_new)
    cn_ref[...] = c_new


def _seg_body(f_ref, rank_ref, uniq_ref, wmt_ref, bm_ref, h_ref,
              mo_ref, so_ref, ro_ref, hloc_ref):
    i = pl.program_id(0)
    tn = f_ref.shape[0]
    hdim = h_ref.shape[1]

    @pl.when(i == 0)
    def _init():
        mo_ref[...] = jnp.full(mo_ref.shape, _NEG, jnp.float32)
        so_ref[...] = jnp.zeros(so_ref.shape, jnp.float32)
        ro_ref[...] = jnp.zeros(ro_ref.shape, jnp.float32)
        hloc_ref[...] = jnp.zeros(hloc_ref.shape, jnp.float32)

    m = (jnp.dot(f_ref[...], wmt_ref[...], preferred_element_type=jnp.float32)
         + bm_ref[...])

    rank_col = rank_ref[0]
    n_seg = rank_ref[0, tn - 1, 0] + 1

    def _gather(j, _):
        g = uniq_ref[0, j, 0]
        hloc_ref[pl.ds(j, 1), :] = h_ref[pl.ds(g, 1), :]
        return 0

    lax.fori_loop(0, n_seg, _gather, 0)

    lane = lax.broadcasted_iota(jnp.int32, (tn, tn), 1)
    onehot = (rank_col == lane).astype(jnp.float32)
    hb = jnp.dot(onehot, hloc_ref[...], preferred_element_type=jnp.float32)
    e = jnp.sum(m * hb, axis=-1, keepdims=True)

    def _merge(j, _):
        g = uniq_ref[0, j, 0]
        mask = rank_col == j
        e_j = jnp.where(mask, e, _NEG)
        m_loc = jnp.max(e_j)
        old_m = jnp.max(mo_ref[pl.ds(g, 1), :])
        new_m = jnp.maximum(old_m, m_loc)
        a = jnp.exp(old_m - new_m)
        b = jnp.exp(m_loc - new_m)
        p = jnp.where(mask, jnp.exp(e - m_loc), 0.0)
        s_j = jnp.sum(p)
        r_j = lax.dot_general(p, m, (((0,), (0,)), ((), ())),
                              preferred_element_type=jnp.float32)
        so_ref[pl.ds(g, 1), :] = so_ref[pl.ds(g, 1), :] * a + s_j * b
        ro_ref[pl.ds(g, 1), :] = ro_ref[pl.ds(g, 1), :] * a + r_j * b
        mo_ref[pl.ds(g, 1), :] = jnp.full((1, mo_ref.shape[1]), 0.0) + new_m
        return 0

    lax.fori_loop(0, n_seg, _merge, 0)


def kernel(features, feature_graph_index, W_m, b_m, W_ih, W_hh, b_ih, b_hh):
    n, d = features.shape
    h_dim = W_hh.shape[1]
    g_num = 10000
    nblk = n // _TN
    seg = feature_graph_index.astype(jnp.int32)

    b0 = jnp.concatenate([jnp.ones((1,), jnp.int32),
                          (seg[1:] != seg[:-1]).astype(jnp.int32)])
    csum = jnp.cumsum(b0)
    blk = jnp.arange(n, dtype=jnp.int32) // _TN
    base = csum[blk * 0 + (blk * _TN)]
    rank = (csum - base).astype(jnp.int32)
    uniq = jnp.zeros((nblk, _TN), jnp.int32).at[blk, rank].set(seg)
    rank3 = rank.reshape(nblk, _TN, 1)
    uniq3 = uniq.reshape(nblk, _TN, 1)

    wmt = W_m.T
    bm2 = b_m.reshape(1, h_dim)
    w_iht = W_ih.T
    wa = w_iht[:h_dim, :] + W_hh.T
    wb = w_iht[h_dim:, :]
    bs = (b_ih + b_hh).reshape(1, 4 * h_dim)

    lstm_call = pl.pallas_call(
        _lstm_body,
        grid=(g_num // _GR,),
        in_specs=[
            pl.BlockSpec((_GR, h_dim), lambda i: (i, 0)),
            pl.BlockSpec((_GR, h_dim), lambda i: (i, 0)),
            pl.BlockSpec((_GR, h_dim), lambda i: (i, 0)),
            pl.BlockSpec((_GR, 128), lambda i: (i, 0)),
            pl.BlockSpec((h_dim, 4 * h_dim), lambda i: (0, 0)),
            pl.BlockSpec((h_dim, 4 * h_dim), lambda i: (0, 0)),
            pl.BlockSpec((1, 4 * h_dim), lambda i: (0, 0)),
        ],
        out_specs=[
            pl.BlockSpec((_GR, h_dim), lambda i: (i, 0)),
            pl.BlockSpec((_GR, h_dim), lambda i: (i, 0)),
        ],
        out_shape=[
            jax.ShapeDtypeStruct((g_num, h_dim), jnp.float32),
            jax.ShapeDtypeStruct((g_num, h_dim), jnp.float32),
        ],
    )

    seg_call = pl.pallas_call(
        _seg_body,
        grid=(nblk,),
        in_specs=[
            pl.BlockSpec((_TN, d), lambda i: (i, 0)),
            pl.BlockSpec((1, _TN, 1), lambda i: (i, 0, 0)),
            pl.BlockSpec((1, _TN, 1), lambda i: (i, 0, 0)),
            pl.BlockSpec((d, h_dim), lambda i: (0, 0)),
            pl.BlockSpec((1, h_dim), lambda i: (0, 0)),
            pl.BlockSpec((g_num, h_dim), lambda i: (0, 0)),
        ],
        out_specs=[
            pl.BlockSpec((g_num, 128), lambda i: (0, 0)),
            pl.BlockSpec((g_num, 128), lambda i: (0, 0)),
            pl.BlockSpec((g_num, h_dim), lambda i: (0, 0)),
        ],
        out_shape=[
            jax.ShapeDtypeStruct((g_num, 128), jnp.float32),
            jax.ShapeDtypeStruct((g_num, 128), jnp.float32),
            jax.ShapeDtypeStruct((g_num, h_dim), jnp.float32),
        ],
        scratch_shapes=[pltpu.VMEM((_TN, h_dim), jnp.float32)],
    )

    h = jnp.zeros((g_num, h_dim), jnp.float32)
    c = jnp.zeros((g_num, h_dim), jnp.float32)
    ro = jnp.zeros((g_num, h_dim), jnp.float32)
    so = jnp.zeros((g_num, 128), jnp.float32)

    for _ in range(_LOOPS):
        h, c = lstm_call(h, c, ro, so, wa, wb, bs)
        _, so, ro = seg_call(features, rank3, uniq3, wmt, bm2, h)

    r_t = ro / (so[:, :1] + _EPSV)
    return jnp.concatenate([h, r_t], axis=-1)


# speedup vs baseline: 1.9265x; 1.9120x over previous
import jax
import jax.numpy as jnp
from jax import lax
from jax.experimental import pallas as pl
from jax.experimental.pallas import tpu as pltpu

_TN = 256
_SG = 256
_GPAD = 10240
_GR = 1024
_LOOPS = 3
_EPSV = 1e-07
_NEG = -1e30


def _lstm_body(ht_ref, ct_ref, rt_ref, s8_ref, wa_ref, wb_ref, bs_ref,
               hn_ref, cn_ref):
    nh = ht_ref.shape[0]
    ht = ht_ref[...]
    ct = ct_ref[...]
    s = s8_ref[:1, :]
    rt = rt_ref[...] / (s + _EPSV)
    gates = (lax.dot_general(wa_ref[...], ht, (((0,), (0,)), ((), ())),
                             preferred_element_type=jnp.float32)
             + lax.dot_general(wb_ref[...], rt, (((0,), (0,)), ((), ())),
                               preferred_element_type=jnp.float32)
             + bs_ref[...])
    i_g = jax.nn.sigmoid(gates[0 * nh:1 * nh, :])
    f_g = jax.nn.sigmoid(gates[1 * nh:2 * nh, :])
    g_g = jnp.tanh(gates[2 * nh:3 * nh, :])
    o_g = jax.nn.sigmoid(gates[3 * nh:4 * nh, :])
    c_new = f_g * ct + i_g * g_g
    hn_ref[...] = o_g * jnp.tanh(c_new)
    cn_ref[...] = c_new


def _seg_body(t0_ref, nt_ref, ht_ref, wmt_ref, bmr_ref, bmc_ref,
              feat_hbm, seg_hbm, rt_ref, s8_ref,
              ftacc_ref, fbuf_ref, gbuf_ref, sem_ref):
    s = pl.program_id(0)
    t0 = t0_ref[s]
    nt = nt_ref[s]
    g0 = s * _SG

    wh = jnp.dot(wmt_ref[...], ht_ref[...], preferred_element_type=jnp.float32)
    eb = jnp.dot(bmr_ref[...], ht_ref[...], preferred_element_type=jnp.float32)

    ftacc_ref[...] = jnp.zeros(ftacc_ref.shape, jnp.float32)

    def _issue(k):
        t = t0 + k
        buf = lax.rem(k, 2)
        pltpu.make_async_copy(feat_hbm.at[pl.ds(t * _TN, _TN), :],
                              fbuf_ref.at[buf], sem_ref.at[buf, 0]).start()
        pltpu.make_async_copy(seg_hbm.at[pl.ds(t * _TN, _TN), :],
                              gbuf_ref.at[buf], sem_ref.at[buf, 1]).start()

    @pl.when(nt > 0)
    def _prime():
        _issue(0)

    lane = lax.broadcasted_iota(jnp.int32, (_TN, _SG), 1)

    def _chunk(k, carry):
        m_run, s_run = carry
        buf = lax.rem(k, 2)

        @pl.when(k + 1 < nt)
        def _prefetch():
            _issue(k + 1)

        t = t0 + k
        pltpu.make_async_copy(feat_hbm.at[pl.ds(t * _TN, _TN), :],
                              fbuf_ref.at[buf], sem_ref.at[buf, 0]).wait()
        pltpu.make_async_copy(seg_hbm.at[pl.ds(t * _TN, _TN), :],
                              gbuf_ref.at[buf], sem_ref.at[buf, 1]).wait()

        f = fbuf_ref[buf]
        rel = gbuf_ref[buf] - g0
        oneb = rel == lane
        inrow = (rel >= 0) & (rel < _SG)

        e_full = jnp.dot(f, wh, preferred_element_type=jnp.float32) + eb
        e_m = jnp.where(oneb, e_full, _NEG)
        cmax = jnp.max(e_m, axis=0, keepdims=True)
        new_m = jnp.maximum(m_run, cmax)
        e_row = jnp.sum(jnp.where(oneb, e_full, 0.0), axis=1, keepdims=True)
        m_row = jnp.sum(jnp.where(oneb, new_m, 0.0), axis=1, keepdims=True)
        p_row = jnp.where(inrow, jnp.exp(e_row - m_row), 0.0)
        p_mat = jnp.where(oneb, p_row, 0.0)
        scale = jnp.exp(m_run - new_m)
        s_new = s_run * scale + jnp.sum(p_mat, axis=0, keepdims=True)
        ftacc_ref[...] = (ftacc_ref[...] * scale
                          + lax.dot_general(f, p_mat, (((0,), (0,)), ((), ())),
                                            preferred_element_type=jnp.float32))
        return new_m, s_new

    m_fin, s_fin = lax.fori_loop(
        0, nt, _chunk,
        (jnp.full((1, _SG), _NEG, jnp.float32),
         jnp.zeros((1, _SG), jnp.float32)))

    rt_ref[...] = (lax.dot_general(wmt_ref[...], ftacc_ref[...],
                                   (((0,), (0,)), ((), ())),
                                   preferred_element_type=jnp.float32)
                   + bmc_ref[...] * s_fin)
    s8_ref[...] = jnp.broadcast_to(s_fin, s8_ref.shape)


def kernel(features, feature_graph_index, W_m, b_m, W_ih, W_hh, b_ih, b_hh):
    n, d = features.shape
    h_dim = W_hh.shape[1]
    seg = feature_graph_index.astype(jnp.int32)
    nstep = _GPAD // _SG

    off = jnp.searchsorted(seg, jnp.arange(_GPAD + 1, dtype=jnp.int32),
                           side='left').astype(jnp.int32)
    off_lo = off[0:_GPAD:_SG]
    off_hi = off[_SG:_GPAD + 1:_SG]
    t0s = off_lo // _TN
    t1s = (off_hi + _TN - 1) // _TN
    nts = jnp.where(off_hi > off_lo, t1s - t0s, 0).astype(jnp.int32)
    t0s = t0s.astype(jnp.int32)

    wmt = W_m.T
    bmr = b_m.reshape(1, h_dim)
    bmc = b_m.reshape(h_dim, 1)
    w_iht = W_ih.T
    wa = w_iht[:h_dim, :] + W_hh.T
    wb = w_iht[h_dim:, :]
    bs = (b_ih + b_hh).reshape(4 * h_dim, 1)
    seg2 = seg.reshape(n, 1)

    lstm_call = pl.pallas_call(
        _lstm_body,
        grid=(_GPAD // _GR,),
        in_specs=[
            pl.BlockSpec((h_dim, _GR), lambda i: (0, i)),
            pl.BlockSpec((h_dim, _GR), lambda i: (0, i)),
            pl.BlockSpec((h_dim, _GR), lambda i: (0, i)),
            pl.BlockSpec((8, _GR), lambda i: (0, i)),
            pl.BlockSpec((h_dim, 4 * h_dim), lambda i: (0, 0)),
            pl.BlockSpec((h_dim, 4 * h_dim), lambda i: (0, 0)),
            pl.BlockSpec((4 * h_dim, 1), lambda i: (0, 0)),
        ],
        out_specs=[
            pl.BlockSpec((h_dim, _GR), lambda i: (0, i)),
            pl.BlockSpec((h_dim, _GR), lambda i: (0, i)),
        ],
        out_shape=[
            jax.ShapeDtypeStruct((h_dim, _GPAD), jnp.float32),
            jax.ShapeDtypeStruct((h_dim, _GPAD), jnp.float32),
        ],
    )

    seg_call = pl.pallas_call(
        _seg_body,
        grid_spec=pltpu.PrefetchScalarGridSpec(
            num_scalar_prefetch=2,
            grid=(nstep,),
            in_specs=[
                pl.BlockSpec((h_dim, _SG), lambda s, t0, nt: (0, s)),
                pl.BlockSpec((d, h_dim), lambda s, t0, nt: (0, 0)),
                pl.BlockSpec((1, h_dim), lambda s, t0, nt: (0, 0)),
                pl.BlockSpec((h_dim, 1), lambda s, t0, nt: (0, 0)),
                pl.BlockSpec(memory_space=pltpu.MemorySpace.HBM),
                pl.BlockSpec(memory_space=pltpu.MemorySpace.HBM),
            ],
            out_specs=[
                pl.BlockSpec((h_dim, _SG), lambda s, t0, nt: (0, s)),
                pl.BlockSpec((8, _SG), lambda s, t0, nt: (0, s)),
            ],
            scratch_shapes=[
                pltpu.VMEM((d, _SG), jnp.float32),
                pltpu.VMEM((2, _TN, d), jnp.float32),
                pltpu.VMEM((2, _TN, 1), jnp.int32),
                pltpu.SemaphoreType.DMA((2, 2)),
            ],
        ),
        out_shape=[
            jax.ShapeDtypeStruct((h_dim, _GPAD), jnp.float32),
            jax.ShapeDtypeStruct((8, _GPAD), jnp.float32),
        ],
    )

    ht = jnp.zeros((h_dim, _GPAD), jnp.float32)
    ct = jnp.zeros((h_dim, _GPAD), jnp.float32)
    rt = jnp.zeros((h_dim, _GPAD), jnp.float32)
    s8 = jnp.zeros((8, _GPAD), jnp.float32)

    for _ in range(_LOOPS):
        ht, ct = lstm_call(ht, ct, rt, s8, wa, wb, bs)
        rt, s8 = seg_call(t0s, nts, ht, wmt, bmr, bmc, features, seg2)

    g_num = 10000
    h_fin = ht[:, :g_num].T
    r_fin = (rt[:, :g_num] / (s8[:1, :g_num] + _EPSV)).T
    return jnp.concatenate([h_fin, r_fin], axis=-1)


# speedup vs baseline: 2.2516x; 1.1687x over previous
import jax
import jax.numpy as jnp
from jax import lax
from jax.experimental import pallas as pl
from jax.experimental.pallas import tpu as pltpu

_TN = 512
_SG = 256
_GPAD = 10240
_GR = 1024
_LOOPS = 3
_EPSV = 1e-07
_NEG = -1e30


def _lstm_body(ht_ref, ct_ref, rt_ref, s8_ref, wa_ref, wb_ref, bs_ref,
               hn_ref, cn_ref):
    nh = ht_ref.shape[0]
    ht = ht_ref[...]
    ct = ct_ref[...]
    s = s8_ref[:1, :]
    rt = rt_ref[...] / (s + _EPSV)
    gates = (lax.dot_general(wa_ref[...], ht, (((0,), (0,)), ((), ())),
                             preferred_element_type=jnp.float32)
             + lax.dot_general(wb_ref[...], rt, (((0,), (0,)), ((), ())),
                               preferred_element_type=jnp.float32)
             + bs_ref[...])
    i_g = jax.nn.sigmoid(gates[0 * nh:1 * nh, :])
    f_g = jax.nn.sigmoid(gates[1 * nh:2 * nh, :])
    g_g = jnp.tanh(gates[2 * nh:3 * nh, :])
    o_g = jax.nn.sigmoid(gates[3 * nh:4 * nh, :])
    c_new = f_g * ct + i_g * g_g
    hn_ref[...] = o_g * jnp.tanh(c_new)
    cn_ref[...] = c_new


def _seg_body(t0_ref, nt_ref, ht_ref, wmt_ref, bmr_ref, bmc_ref,
              feat_hbm, seg_hbm, rt_ref, s8_ref,
              ftacc_ref, fbuf_ref, gbuf_ref, sem_ref):
    s = pl.program_id(0)
    t0 = t0_ref[s]
    nt = nt_ref[s]
    g0 = s * _SG

    wh = jnp.dot(wmt_ref[...], ht_ref[...], preferred_element_type=jnp.float32)
    eb = jnp.dot(bmr_ref[...], ht_ref[...], preferred_element_type=jnp.float32)

    ftacc_ref[...] = jnp.zeros(ftacc_ref.shape, jnp.float32)

    def _issue(k):
        t = t0 + k
        buf = lax.rem(k, 2)
        pltpu.make_async_copy(feat_hbm.at[pl.ds(t * _TN, _TN), :],
                              fbuf_ref.at[buf], sem_ref.at[buf, 0]).start()
        pltpu.make_async_copy(seg_hbm.at[pl.ds(t * _TN, _TN), :],
                              gbuf_ref.at[buf], sem_ref.at[buf, 1]).start()

    @pl.when(nt > 0)
    def _prime():
        _issue(0)

    lane = lax.broadcasted_iota(jnp.int32, (_TN, _SG), 1)

    def _chunk(k, carry):
        m_run, s_run = carry
        buf = lax.rem(k, 2)

        @pl.when(k + 1 < nt)
        def _prefetch():
            _issue(k + 1)

        t = t0 + k
        pltpu.make_async_copy(feat_hbm.at[pl.ds(t * _TN, _TN), :],
                              fbuf_ref.at[buf], sem_ref.at[buf, 0]).wait()
        pltpu.make_async_copy(seg_hbm.at[pl.ds(t * _TN, _TN), :],
                              gbuf_ref.at[buf], sem_ref.at[buf, 1]).wait()

        f = fbuf_ref[buf]
        rel = gbuf_ref[buf] - g0
        oneb = rel == lane

        e_full = jnp.dot(f, wh, preferred_element_type=jnp.float32) + eb
        e_m = jnp.where(oneb, e_full, _NEG)
        cmax = jnp.max(e_m, axis=0, keepdims=True)
        new_m = jnp.maximum(m_run, cmax)
        p_mat = jnp.where(oneb, jnp.exp(e_full - new_m), 0.0)
        scale = jnp.exp(m_run - new_m)
        s_new = s_run * scale + jnp.sum(p_mat, axis=0, keepdims=True)
        ftacc_ref[...] = (ftacc_ref[...] * scale
                          + lax.dot_general(f, p_mat, (((0,), (0,)), ((), ())),
                                            preferred_element_type=jnp.float32))
        return new_m, s_new

    m_fin, s_fin = lax.fori_loop(
        0, nt, _chunk,
        (jnp.full((1, _SG), _NEG, jnp.float32),
         jnp.zeros((1, _SG), jnp.float32)))

    rt_ref[...] = (lax.dot_general(wmt_ref[...], ftacc_ref[...],
                                   (((0,), (0,)), ((), ())),
                                   preferred_element_type=jnp.float32)
                   + bmc_ref[...] * s_fin)
    s8_ref[...] = jnp.broadcast_to(s_fin, s8_ref.shape)


def kernel(features, feature_graph_index, W_m, b_m, W_ih, W_hh, b_ih, b_hh):
    n, d = features.shape
    h_dim = W_hh.shape[1]
    seg = feature_graph_index.astype(jnp.int32)
    nstep = _GPAD // _SG

    off = jnp.searchsorted(seg, jnp.arange(_GPAD + 1, dtype=jnp.int32),
                           side='left').astype(jnp.int32)
    off_lo = off[0:_GPAD:_SG]
    off_hi = off[_SG:_GPAD + 1:_SG]
    t0s = off_lo // _TN
    t1s = (off_hi + _TN - 1) // _TN
    nts = jnp.where(off_hi > off_lo, t1s - t0s, 0).astype(jnp.int32)
    t0s = t0s.astype(jnp.int32)

    wmt = W_m.T
    bmr = b_m.reshape(1, h_dim)
    bmc = b_m.reshape(h_dim, 1)
    w_iht = W_ih.T
    wa = w_iht[:h_dim, :] + W_hh.T
    wb = w_iht[h_dim:, :]
    bs = (b_ih + b_hh).reshape(4 * h_dim, 1)
    seg2 = seg.reshape(n, 1)

    lstm_call = pl.pallas_call(
        _lstm_body,
        grid=(_GPAD // _GR,),
        in_specs=[
            pl.BlockSpec((h_dim, _GR), lambda i: (0, i)),
            pl.BlockSpec((h_dim, _GR), lambda i: (0, i)),
            pl.BlockSpec((h_dim, _GR), lambda i: (0, i)),
            pl.BlockSpec((8, _GR), lambda i: (0, i)),
            pl.BlockSpec((h_dim, 4 * h_dim), lambda i: (0, 0)),
            pl.BlockSpec((h_dim, 4 * h_dim), lambda i: (0, 0)),
            pl.BlockSpec((4 * h_dim, 1), lambda i: (0, 0)),
        ],
        out_specs=[
            pl.BlockSpec((h_dim, _GR), lambda i: (0, i)),
            pl.BlockSpec((h_dim, _GR), lambda i: (0, i)),
        ],
        out_shape=[
            jax.ShapeDtypeStruct((h_dim, _GPAD), jnp.float32),
            jax.ShapeDtypeStruct((h_dim, _GPAD), jnp.float32),
        ],
    )

    seg_call = pl.pallas_call(
        _seg_body,
        grid_spec=pltpu.PrefetchScalarGridSpec(
            num_scalar_prefetch=2,
            grid=(nstep,),
            in_specs=[
                pl.BlockSpec((h_dim, _SG), lambda s, t0, nt: (0, s)),
                pl.BlockSpec((d, h_dim), lambda s, t0, nt: (0, 0)),
                pl.BlockSpec((1, h_dim), lambda s, t0, nt: (0, 0)),
                pl.BlockSpec((h_dim, 1), lambda s, t0, nt: (0, 0)),
                pl.BlockSpec(memory_space=pltpu.MemorySpace.HBM),
                pl.BlockSpec(memory_space=pltpu.MemorySpace.HBM),
            ],
            out_specs=[
                pl.BlockSpec((h_dim, _SG), lambda s, t0, nt: (0, s)),
                pl.BlockSpec((8, _SG), lambda s, t0, nt: (0, s)),
            ],
            scratch_shapes=[
                pltpu.VMEM((d, _SG), jnp.float32),
                pltpu.VMEM((2, _TN, d), jnp.float32),
                pltpu.VMEM((2, _TN, 1), jnp.int32),
                pltpu.SemaphoreType.DMA((2, 2)),
            ],
        ),
        out_shape=[
            jax.ShapeDtypeStruct((h_dim, _GPAD), jnp.float32),
            jax.ShapeDtypeStruct((8, _GPAD), jnp.float32),
        ],
    )

    ht = jnp.zeros((h_dim, _GPAD), jnp.float32)
    ct = jnp.zeros((h_dim, _GPAD), jnp.float32)
    rt = jnp.zeros((h_dim, _GPAD), jnp.float32)
    s8 = jnp.zeros((8, _GPAD), jnp.float32)

    for _ in range(_LOOPS):
        ht, ct = lstm_call(ht, ct, rt, s8, wa, wb, bs)
        rt, s8 = seg_call(t0s, nts, ht, wmt, bmr, bmc, features, seg2)

    g_num = 10000
    h_fin = ht[:, :g_num].T
    r_fin = (rt[:, :g_num] / (s8[:1, :g_num] + _EPSV)).T
    return jnp.concatenate([h_fin, r_fin], axis=-1)


# speedup vs baseline: 2.3884x; 1.0607x over previous
import jax
import jax.numpy as jnp
from jax import lax
from jax.experimental import pallas as pl
from jax.experimental.pallas import tpu as pltpu

_TN = 512
_SG = 256
_GPAD = 10240
_GR = 1024
_NB = 8
_LOOPS = 3
_EPSV = 1e-07
_NEG = -1e30


def _lstm_body(ht_ref, ct_ref, rt_ref, s8_ref, wa_ref, wb_ref, bs_ref,
               hn_ref, cn_ref):
    nh = ht_ref.shape[0]
    ht = ht_ref[...]
    ct = ct_ref[...]
    s = s8_ref[:1, :]
    rt = rt_ref[...] / (s + _EPSV)
    gates = (lax.dot_general(wa_ref[...], ht, (((0,), (0,)), ((), ())),
                             preferred_element_type=jnp.float32)
             + lax.dot_general(wb_ref[...], rt, (((0,), (0,)), ((), ())),
                               preferred_element_type=jnp.float32)
             + bs_ref[...])
    i_g = jax.nn.sigmoid(gates[0 * nh:1 * nh, :])
    f_g = jax.nn.sigmoid(gates[1 * nh:2 * nh, :])
    g_g = jnp.tanh(gates[2 * nh:3 * nh, :])
    o_g = jax.nn.sigmoid(gates[3 * nh:4 * nh, :])
    c_new = f_g * ct + i_g * g_g
    hn_ref[...] = o_g * jnp.tanh(c_new)
    cn_ref[...] = c_new


def _seg_body(t0_ref, nt_ref, ht_ref, wmt_ref, bmr_ref, bmc_ref,
              feat_hbm, seg_hbm, rt_ref, s8_ref,
              ftacc_ref, fbuf_ref, gbuf_ref, sem_ref):
    s = pl.program_id(0)
    t0 = t0_ref[s]
    nt = nt_ref[s]
    g0 = s * _SG

    wh = jnp.dot(wmt_ref[...], ht_ref[...], preferred_element_type=jnp.float32)
    eb = jnp.dot(bmr_ref[...], ht_ref[...], preferred_element_type=jnp.float32)

    ftacc_ref[...] = jnp.zeros(ftacc_ref.shape, jnp.float32)

    def _issue(k):
        t = t0 + k
        buf = lax.rem(k, _NB)
        pltpu.make_async_copy(feat_hbm.at[pl.ds(t * _TN, _TN), :],
                              fbuf_ref.at[buf], sem_ref.at[buf, 0]).start()
        pltpu.make_async_copy(seg_hbm.at[pl.ds(t * _TN, _TN), :],
                              gbuf_ref.at[buf], sem_ref.at[buf, 1]).start()

    def _prime(k, _):
        _issue(k)
        return 0

    lax.fori_loop(0, jnp.minimum(nt, _NB - 1), _prime, 0)

    lane = lax.broadcasted_iota(jnp.int32, (_TN, _SG), 1)

    def _chunk(k, carry):
        m_run, s_run = carry
        buf = lax.rem(k, _NB)

        @pl.when(k + _NB - 1 < nt)
        def _prefetch():
            _issue(k + _NB - 1)

        t = t0 + k
        pltpu.make_async_copy(feat_hbm.at[pl.ds(t * _TN, _TN), :],
                              fbuf_ref.at[buf], sem_ref.at[buf, 0]).wait()
        pltpu.make_async_copy(seg_hbm.at[pl.ds(t * _TN, _TN), :],
                              gbuf_ref.at[buf], sem_ref.at[buf, 1]).wait()

        f = fbuf_ref[buf]
        rel = gbuf_ref[buf] - g0
        oneb = rel == lane

        e_full = jnp.dot(f, wh, preferred_element_type=jnp.float32) + eb
        e_m = jnp.where(oneb, e_full, _NEG)
        cmax = jnp.max(e_m, axis=0, keepdims=True)
        new_m = jnp.maximum(m_run, cmax)
        p_mat = jnp.where(oneb, jnp.exp(e_full - new_m), 0.0)
        scale = jnp.exp(m_run - new_m)
        s_new = s_run * scale + jnp.sum(p_mat, axis=0, keepdims=True)
        ftacc_ref[...] = (ftacc_ref[...] * scale
                          + lax.dot_general(f, p_mat, (((0,), (0,)), ((), ())),
                                            preferred_element_type=jnp.float32))
        return new_m, s_new

    m_fin, s_fin = lax.fori_loop(
        0, nt, _chunk,
        (jnp.full((1, _SG), _NEG, jnp.float32),
         jnp.zeros((1, _SG), jnp.float32)))

    rt_ref[...] = (lax.dot_general(wmt_ref[...], ftacc_ref[...],
                                   (((0,), (0,)), ((), ())),
                                   preferred_element_type=jnp.float32)
                   + bmc_ref[...] * s_fin)
    s8_ref[...] = jnp.broadcast_to(s_fin, s8_ref.shape)


def kernel(features, feature_graph_index, W_m, b_m, W_ih, W_hh, b_ih, b_hh):
    n, d = features.shape
    h_dim = W_hh.shape[1]
    seg = feature_graph_index.astype(jnp.int32)
    nstep = _GPAD // _SG

    off = jnp.searchsorted(seg, jnp.arange(_GPAD + 1, dtype=jnp.int32),
                           side='left').astype(jnp.int32)
    off_lo = off[0:_GPAD:_SG]
    off_hi = off[_SG:_GPAD + 1:_SG]
    t0s = off_lo // _TN
    t1s = (off_hi + _TN - 1) // _TN
    nts = jnp.where(off_hi > off_lo, t1s - t0s, 0).astype(jnp.int32)
    t0s = t0s.astype(jnp.int32)

    wmt = W_m.T
    bmr = b_m.reshape(1, h_dim)
    bmc = b_m.reshape(h_dim, 1)
    w_iht = W_ih.T
    wa = w_iht[:h_dim, :] + W_hh.T
    wb = w_iht[h_dim:, :]
    bs = (b_ih + b_hh).reshape(4 * h_dim, 1)
    seg2 = seg.reshape(n, 1)

    lstm_call = pl.pallas_call(
        _lstm_body,
        grid=(_GPAD // _GR,),
        in_specs=[
            pl.BlockSpec((h_dim, _GR), lambda i: (0, i)),
            pl.BlockSpec((h_dim, _GR), lambda i: (0, i)),
            pl.BlockSpec((h_dim, _GR), lambda i: (0, i)),
            pl.BlockSpec((8, _GR), lambda i: (0, i)),
            pl.BlockSpec((h_dim, 4 * h_dim), lambda i: (0, 0)),
            pl.BlockSpec((h_dim, 4 * h_dim), lambda i: (0, 0)),
            pl.BlockSpec((4 * h_dim, 1), lambda i: (0, 0)),
        ],
        out_specs=[
            pl.BlockSpec((h_dim, _GR), lambda i: (0, i)),
            pl.BlockSpec((h_dim, _GR), lambda i: (0, i)),
        ],
        out_shape=[
            jax.ShapeDtypeStruct((h_dim, _GPAD), jnp.float32),
            jax.ShapeDtypeStruct((h_dim, _GPAD), jnp.float32),
        ],
    )

    seg_call = pl.pallas_call(
        _seg_body,
        grid_spec=pltpu.PrefetchScalarGridSpec(
            num_scalar_prefetch=2,
            grid=(nstep,),
            in_specs=[
                pl.BlockSpec((h_dim, _SG), lambda s, t0, nt: (0, s)),
                pl.BlockSpec((d, h_dim), lambda s, t0, nt: (0, 0)),
                pl.BlockSpec((1, h_dim), lambda s, t0, nt: (0, 0)),
                pl.BlockSpec((h_dim, 1), lambda s, t0, nt: (0, 0)),
                pl.BlockSpec(memory_space=pltpu.MemorySpace.HBM),
                pl.BlockSpec(memory_space=pltpu.MemorySpace.HBM),
            ],
            out_specs=[
                pl.BlockSpec((h_dim, _SG), lambda s, t0, nt: (0, s)),
                pl.BlockSpec((8, _SG), lambda s, t0, nt: (0, s)),
            ],
            scratch_shapes=[
                pltpu.VMEM((d, _SG), jnp.float32),
                pltpu.VMEM((_NB, _TN, d), jnp.float32),
                pltpu.VMEM((_NB, _TN, 1), jnp.int32),
                pltpu.SemaphoreType.DMA((_NB, 2)),
            ],
        ),
        out_shape=[
            jax.ShapeDtypeStruct((h_dim, _GPAD), jnp.float32),
            jax.ShapeDtypeStruct((8, _GPAD), jnp.float32),
        ],
    )

    ht = jnp.zeros((h_dim, _GPAD), jnp.float32)
    ct = jnp.zeros((h_dim, _GPAD), jnp.float32)
    rt = jnp.zeros((h_dim, _GPAD), jnp.float32)
    s8 = jnp.zeros((8, _GPAD), jnp.float32)

    for _ in range(_LOOPS):
        ht, ct = lstm_call(ht, ct, rt, s8, wa, wb, bs)
        rt, s8 = seg_call(t0s, nts, ht, wmt, bmr, bmc, features, seg2)

    g_num = 10000
    h_fin = ht[:, :g_num].T
    r_fin = (rt[:, :g_num] / (s8[:1, :g_num] + _EPSV)).T
    return jnp.concatenate([h_fin, r_fin], axis=-1)


# speedup vs baseline: 10.7535x; 4.5025x over previous
import jax
import jax.numpy as jnp
from jax import lax
from jax.experimental import pallas as pl
from jax.experimental.pallas import tpu as pltpu

_TN = 512
_SG = 256
_GPAD = 10240
_GR = 1024
_NB = 8
_LOOPS = 3
_EPSV = 1e-07
_NEG = -1e30


def _lstm_body(ht_ref, ct_ref, rt_ref, s8_ref, wa_ref, wb_ref, bs_ref,
               hn_ref, cn_ref):
    nh = ht_ref.shape[0]
    ht = ht_ref[...]
    ct = ct_ref[...]
    s = s8_ref[:1, :]
    rt = rt_ref[...] / (s + _EPSV)
    gates = (lax.dot_general(wa_ref[...], ht, (((0,), (0,)), ((), ())),
                             preferred_element_type=jnp.float32)
             + lax.dot_general(wb_ref[...], rt, (((0,), (0,)), ((), ())),
                               preferred_element_type=jnp.float32)
             + bs_ref[...])
    i_g = jax.nn.sigmoid(gates[0 * nh:1 * nh, :])
    f_g = jax.nn.sigmoid(gates[1 * nh:2 * nh, :])
    g_g = jnp.tanh(gates[2 * nh:3 * nh, :])
    o_g = jax.nn.sigmoid(gates[3 * nh:4 * nh, :])
    c_new = f_g * ct + i_g * g_g
    hn_ref[...] = o_g * jnp.tanh(c_new)
    cn_ref[...] = c_new


def _seg_body(t0_ref, nt_ref, ht_ref, wmt_ref, bmr_ref, bmc_ref,
              feat_hbm, seg_hbm, rt_ref, s8_ref,
              ftacc_ref, fbuf_ref, gbuf_ref, sem_ref):
    s = pl.program_id(0)
    t0 = t0_ref[s]
    nt = nt_ref[s]
    g0 = s * _SG

    wh = jnp.dot(wmt_ref[...], ht_ref[...], preferred_element_type=jnp.float32)
    eb = jnp.dot(bmr_ref[...], ht_ref[...], preferred_element_type=jnp.float32)

    ftacc_ref[...] = jnp.zeros(ftacc_ref.shape, jnp.float32)

    def _issue(k):
        t = t0 + k
        buf = lax.rem(k, _NB)
        pltpu.make_async_copy(feat_hbm.at[pl.ds(t * _TN, _TN), :],
                              fbuf_ref.at[buf], sem_ref.at[buf, 0]).start()
        pltpu.make_async_copy(seg_hbm.at[pl.ds(t * _TN, _TN), :],
                              gbuf_ref.at[buf], sem_ref.at[buf, 1]).start()

    def _prime(k, _):
        _issue(k)
        return 0

    lax.fori_loop(0, jnp.minimum(nt, _NB - 1), _prime, 0)

    lane = lax.broadcasted_iota(jnp.int32, (_TN, _SG), 1)

    def _chunk(k, carry):
        m_run, s_run = carry
        buf = lax.rem(k, _NB)

        @pl.when(k + _NB - 1 < nt)
        def _prefetch():
            _issue(k + _NB - 1)

        t = t0 + k
        pltpu.make_async_copy(feat_hbm.at[pl.ds(t * _TN, _TN), :],
                              fbuf_ref.at[buf], sem_ref.at[buf, 0]).wait()
        pltpu.make_async_copy(seg_hbm.at[pl.ds(t * _TN, _TN), :],
                              gbuf_ref.at[buf], sem_ref.at[buf, 1]).wait()

        f = fbuf_ref[buf]
        rel = gbuf_ref[buf] - g0
        oneb = rel == lane

        e_full = jnp.dot(f, wh, preferred_element_type=jnp.float32) + eb
        e_m = jnp.where(oneb, e_full, _NEG)
        cmax = jnp.max(e_m, axis=0, keepdims=True)
        new_m = jnp.maximum(m_run, cmax)
        p_mat = jnp.where(oneb, jnp.exp(e_full - new_m), 0.0)
        scale = jnp.exp(m_run - new_m)
        s_new = s_run * scale + jnp.sum(p_mat, axis=0, keepdims=True)
        ftacc_ref[...] = (ftacc_ref[...] * scale
                          + lax.dot_general(f, p_mat, (((0,), (0,)), ((), ())),
                                            preferred_element_type=jnp.float32))
        return new_m, s_new

    m_fin, s_fin = lax.fori_loop(
        0, nt, _chunk,
        (jnp.full((1, _SG), _NEG, jnp.float32),
         jnp.zeros((1, _SG), jnp.float32)))

    rt_ref[...] = (lax.dot_general(wmt_ref[...], ftacc_ref[...],
                                   (((0,), (0,)), ((), ())),
                                   preferred_element_type=jnp.float32)
                   + bmc_ref[...] * s_fin)
    s8_ref[...] = jnp.broadcast_to(s_fin, s8_ref.shape)


def kernel(features, feature_graph_index, W_m, b_m, W_ih, W_hh, b_ih, b_hh):
    n, d = features.shape
    h_dim = W_hh.shape[1]
    seg = feature_graph_index.astype(jnp.int32)
    nstep = _GPAD // _SG

    bounds = jnp.arange(0, _GPAD + 1, _SG, dtype=jnp.int32)
    off = jnp.sum((seg[:, None] < bounds[None, :]).astype(jnp.int32),
                  axis=0).astype(jnp.int32)
    off_lo = off[:-1]
    off_hi = off[1:]
    t0s = off_lo // _TN
    t1s = (off_hi + _TN - 1) // _TN
    nts = jnp.where(off_hi > off_lo, t1s - t0s, 0).astype(jnp.int32)
    t0s = t0s.astype(jnp.int32)

    wmt = W_m.T
    bmr = b_m.reshape(1, h_dim)
    bmc = b_m.reshape(h_dim, 1)
    w_iht = W_ih.T
    wa = w_iht[:h_dim, :] + W_hh.T
    wb = w_iht[h_dim:, :]
    bs = (b_ih + b_hh).reshape(4 * h_dim, 1)
    seg2 = seg.reshape(n, 1)

    lstm_call = pl.pallas_call(
        _lstm_body,
        grid=(_GPAD // _GR,),
        in_specs=[
            pl.BlockSpec((h_dim, _GR), lambda i: (0, i)),
            pl.BlockSpec((h_dim, _GR), lambda i: (0, i)),
            pl.BlockSpec((h_dim, _GR), lambda i: (0, i)),
            pl.BlockSpec((8, _GR), lambda i: (0, i)),
            pl.BlockSpec((h_dim, 4 * h_dim), lambda i: (0, 0)),
            pl.BlockSpec((h_dim, 4 * h_dim), lambda i: (0, 0)),
            pl.BlockSpec((4 * h_dim, 1), lambda i: (0, 0)),
        ],
        out_specs=[
            pl.BlockSpec((h_dim, _GR), lambda i: (0, i)),
            pl.BlockSpec((h_dim, _GR), lambda i: (0, i)),
        ],
        out_shape=[
            jax.ShapeDtypeStruct((h_dim, _GPAD), jnp.float32),
            jax.ShapeDtypeStruct((h_dim, _GPAD), jnp.float32),
        ],
    )

    seg_call = pl.pallas_call(
        _seg_body,
        grid_spec=pltpu.PrefetchScalarGridSpec(
            num_scalar_prefetch=2,
            grid=(nstep,),
            in_specs=[
                pl.BlockSpec((h_dim, _SG), lambda s, t0, nt: (0, s)),
                pl.BlockSpec((d, h_dim), lambda s, t0, nt: (0, 0)),
                pl.BlockSpec((1, h_dim), lambda s, t0, nt: (0, 0)),
                pl.BlockSpec((h_dim, 1), lambda s, t0, nt: (0, 0)),
                pl.BlockSpec(memory_space=pltpu.MemorySpace.HBM),
                pl.BlockSpec(memory_space=pltpu.MemorySpace.HBM),
            ],
            out_specs=[
                pl.BlockSpec((h_dim, _SG), lambda s, t0, nt: (0, s)),
                pl.BlockSpec((8, _SG), lambda s, t0, nt: (0, s)),
            ],
            scratch_shapes=[
                pltpu.VMEM((d, _SG), jnp.float32),
                pltpu.VMEM((_NB, _TN, d), jnp.float32),
                pltpu.VMEM((_NB, _TN, 1), jnp.int32),
                pltpu.SemaphoreType.DMA((_NB, 2)),
            ],
        ),
        out_shape=[
            jax.ShapeDtypeStruct((h_dim, _GPAD), jnp.float32),
            jax.ShapeDtypeStruct((8, _GPAD), jnp.float32),
        ],
    )

    ht = jnp.zeros((h_dim, _GPAD), jnp.float32)
    ct = jnp.zeros((h_dim, _GPAD), jnp.float32)
    rt = jnp.zeros((h_dim, _GPAD), jnp.float32)
    s8 = jnp.zeros((8, _GPAD), jnp.float32)

    for _ in range(_LOOPS):
        ht, ct = lstm_call(ht, ct, rt, s8, wa, wb, bs)
        rt, s8 = seg_call(t0s, nts, ht, wmt, bmr, bmc, features, seg2)

    g_num = 10000
    h_fin = ht[:, :g_num].T
    r_fin = (rt[:, :g_num] / (s8[:1, :g_num] + _EPSV)).T
    return jnp.concatenate([h_fin, r_fin], axis=-1)


# speedup vs baseline: 12.1187x; 1.1270x over previous
import jax
import jax.numpy as jnp
from jax import lax
from jax.experimental import pallas as pl
from jax.experimental.pallas import tpu as pltpu

_TN = 1024
_SG = 256
_GPAD = 10240
_GR = 1024
_NB = 8
_LOOPS = 3
_EPSV = 1e-07
_NEG = -1e30


def _lstm_body(ht_ref, ct_ref, rt_ref, s8_ref, wa_ref, wb_ref, bs_ref,
               hn_ref, cn_ref):
    nh = ht_ref.shape[0]
    ht = ht_ref[...]
    ct = ct_ref[...]
    s = s8_ref[:1, :]
    rt = rt_ref[...] / (s + _EPSV)
    gates = (lax.dot_general(wa_ref[...], ht, (((0,), (0,)), ((), ())),
                             preferred_element_type=jnp.float32)
             + lax.dot_general(wb_ref[...], rt, (((0,), (0,)), ((), ())),
                               preferred_element_type=jnp.float32)
             + bs_ref[...])
    i_g = jax.nn.sigmoid(gates[0 * nh:1 * nh, :])
    f_g = jax.nn.sigmoid(gates[1 * nh:2 * nh, :])
    g_g = jnp.tanh(gates[2 * nh:3 * nh, :])
    o_g = jax.nn.sigmoid(gates[3 * nh:4 * nh, :])
    c_new = f_g * ct + i_g * g_g
    hn_ref[...] = o_g * jnp.tanh(c_new)
    cn_ref[...] = c_new


def _seg_body(t0_ref, nt_ref, ht_ref, wmt_ref, bmr_ref, bmc_ref,
              feat_hbm, seg_hbm, rt_ref, s8_ref,
              ftacc_ref, fbuf_ref, gbuf_ref, sem_ref):
    s = pl.program_id(0)
    t0 = t0_ref[s]
    nt = nt_ref[s]
    g0 = s * _SG

    wh = jnp.dot(wmt_ref[...], ht_ref[...], preferred_element_type=jnp.float32)
    eb = jnp.dot(bmr_ref[...], ht_ref[...], preferred_element_type=jnp.float32)

    ftacc_ref[...] = jnp.zeros(ftacc_ref.shape, jnp.float32)

    def _issue(k):
        t = t0 + k
        buf = lax.rem(k, _NB)
        pltpu.make_async_copy(feat_hbm.at[pl.ds(t * _TN, _TN), :],
                              fbuf_ref.at[buf], sem_ref.at[buf, 0]).start()
        pltpu.make_async_copy(seg_hbm.at[pl.ds(t * _TN, _TN), :],
                              gbuf_ref.at[buf], sem_ref.at[buf, 1]).start()

    def _prime(k, _):
        _issue(k)
        return 0

    lax.fori_loop(0, jnp.minimum(nt, _NB - 1), _prime, 0)

    lane = lax.broadcasted_iota(jnp.int32, (_TN, _SG), 1)

    def _chunk(k, carry):
        m_run, s_run = carry
        buf = lax.rem(k, _NB)

        @pl.when(k + _NB - 1 < nt)
        def _prefetch():
            _issue(k + _NB - 1)

        t = t0 + k
        pltpu.make_async_copy(feat_hbm.at[pl.ds(t * _TN, _TN), :],
                              fbuf_ref.at[buf], sem_ref.at[buf, 0]).wait()
        pltpu.make_async_copy(seg_hbm.at[pl.ds(t * _TN, _TN), :],
                              gbuf_ref.at[buf], sem_ref.at[buf, 1]).wait()

        f = fbuf_ref[buf]
        rel = gbuf_ref[buf] - g0
        oneb = rel == lane

        e_full = jnp.dot(f, wh, preferred_element_type=jnp.float32) + eb
        e_m = jnp.where(oneb, e_full, _NEG)
        cmax = jnp.max(e_m, axis=0, keepdims=True)
        new_m = jnp.maximum(m_run, cmax)
        p_mat = jnp.where(oneb, jnp.exp(e_full - new_m), 0.0)
        scale = jnp.exp(m_run - new_m)
        s_new = s_run * scale + jnp.sum(p_mat, axis=0, keepdims=True)
        ftacc_ref[...] = (ftacc_ref[...] * scale
                          + lax.dot_general(f, p_mat, (((0,), (0,)), ((), ())),
                                            preferred_element_type=jnp.float32))
        return new_m, s_new

    m_fin, s_fin = lax.fori_loop(
        0, nt, _chunk,
        (jnp.full((1, _SG), _NEG, jnp.float32),
         jnp.zeros((1, _SG), jnp.float32)))

    rt_ref[...] = (lax.dot_general(wmt_ref[...], ftacc_ref[...],
                                   (((0,), (0,)), ((), ())),
                                   preferred_element_type=jnp.float32)
                   + bmc_ref[...] * s_fin)
    s8_ref[...] = jnp.broadcast_to(s_fin, s8_ref.shape)


def kernel(features, feature_graph_index, W_m, b_m, W_ih, W_hh, b_ih, b_hh):
    n, d = features.shape
    h_dim = W_hh.shape[1]
    seg = feature_graph_index.astype(jnp.int32)
    nstep = _GPAD // _SG

    bounds = jnp.arange(0, _GPAD + 1, _SG, dtype=jnp.int32)
    off = jnp.sum((seg[:, None] < bounds[None, :]).astype(jnp.int32),
                  axis=0).astype(jnp.int32)
    off_lo = off[:-1]
    off_hi = off[1:]
    t0s = off_lo // _TN
    t1s = (off_hi + _TN - 1) // _TN
    nts = jnp.where(off_hi > off_lo, t1s - t0s, 0).astype(jnp.int32)
    t0s = t0s.astype(jnp.int32)

    wmt = W_m.T
    bmr = b_m.reshape(1, h_dim)
    bmc = b_m.reshape(h_dim, 1)
    w_iht = W_ih.T
    wa = w_iht[:h_dim, :] + W_hh.T
    wb = w_iht[h_dim:, :]
    bs = (b_ih + b_hh).reshape(4 * h_dim, 1)
    seg2 = seg.reshape(n, 1)

    lstm_call = pl.pallas_call(
        _lstm_body,
        grid=(_GPAD // _GR,),
        in_specs=[
            pl.BlockSpec((h_dim, _GR), lambda i: (0, i)),
            pl.BlockSpec((h_dim, _GR), lambda i: (0, i)),
            pl.BlockSpec((h_dim, _GR), lambda i: (0, i)),
            pl.BlockSpec((8, _GR), lambda i: (0, i)),
            pl.BlockSpec((h_dim, 4 * h_dim), lambda i: (0, 0)),
            pl.BlockSpec((h_dim, 4 * h_dim), lambda i: (0, 0)),
            pl.BlockSpec((4 * h_dim, 1), lambda i: (0, 0)),
        ],
        out_specs=[
            pl.BlockSpec((h_dim, _GR), lambda i: (0, i)),
            pl.BlockSpec((h_dim, _GR), lambda i: (0, i)),
        ],
        out_shape=[
            jax.ShapeDtypeStruct((h_dim, _GPAD), jnp.float32),
            jax.ShapeDtypeStruct((h_dim, _GPAD), jnp.float32),
        ],
    )

    seg_call = pl.pallas_call(
        _seg_body,
        grid_spec=pltpu.PrefetchScalarGridSpec(
            num_scalar_prefetch=2,
            grid=(nstep,),
            in_specs=[
                pl.BlockSpec((h_dim, _SG), lambda s, t0, nt: (0, s)),
                pl.BlockSpec((d, h_dim), lambda s, t0, nt: (0, 0)),
                pl.BlockSpec((1, h_dim), lambda s, t0, nt: (0, 0)),
                pl.BlockSpec((h_dim, 1), lambda s, t0, nt: (0, 0)),
                pl.BlockSpec(memory_space=pltpu.MemorySpace.HBM),
                pl.BlockSpec(memory_space=pltpu.MemorySpace.HBM),
            ],
            out_specs=[
                pl.BlockSpec((h_dim, _SG), lambda s, t0, nt: (0, s)),
                pl.BlockSpec((8, _SG), lambda s, t0, nt: (0, s)),
            ],
            scratch_shapes=[
                pltpu.VMEM((d, _SG), jnp.float32),
                pltpu.VMEM((_NB, _TN, d), jnp.float32),
                pltpu.VMEM((_NB, _TN, 1), jnp.int32),
                pltpu.SemaphoreType.DMA((_NB, 2)),
            ],
        ),
        out_shape=[
            jax.ShapeDtypeStruct((h_dim, _GPAD), jnp.float32),
            jax.ShapeDtypeStruct((8, _GPAD), jnp.float32),
        ],
    )

    ht = jnp.zeros((h_dim, _GPAD), jnp.float32)
    ct = jnp.zeros((h_dim, _GPAD), jnp.float32)
    rt = jnp.zeros((h_dim, _GPAD), jnp.float32)
    s8 = jnp.zeros((8, _GPAD), jnp.float32)

    for _ in range(_LOOPS):
        ht, ct = lstm_call(ht, ct, rt, s8, wa, wb, bs)
        rt, s8 = seg_call(t0s, nts, ht, wmt, bmr, bmc, features, seg2)

    g_num = 10000
    h_fin = ht[:, :g_num].T
    r_fin = (rt[:, :g_num] / (s8[:1, :g_num] + _EPSV)).T
    return jnp.concatenate([h_fin, r_fin], axis=-1)
